# Initial kernel scaffold; baseline (speedup 1.0000x reference)
import jax
import jax.numpy as jnp
from jax.experimental import pallas as pl


def kernel(node_attr, edge_index, edge_attr, W1e, b1e, W2e, b2e, W1n, b1n, W2n, b2n):
    raise NotImplementedError("write your pallas kernel here")


# speedup vs baseline: 3.4159x; 3.4159x over previous
import functools

import jax
import jax.numpy as jnp
from jax import lax
from jax.experimental import pallas as pl
from jax.experimental.pallas import tpu as pltpu
from jax.experimental.pallas import tpu_sc as plsc

N = 10000
E = 320000
D = 128

NC = 2
NS = 16
NW = NC * NS
B = 128
NB = E // B
ITERS = (NB + NW - 1) // NW


def _pre_body(node_ref, w1s_ref, w1r_ref, ps_ref, pr_ref):
  x = node_ref[...]
  ps_ref[...] = jnp.dot(x, w1s_ref[...], preferred_element_type=jnp.float32)
  pr_ref[...] = jnp.dot(x, w1r_ref[...], preferred_element_type=jnp.float32)


def _edge_body(gs_ref, gr_ref, e_ref, w1x_ref, b1_ref, w2_ref, b2_ref,
               enew_ref, eout_ref):
  e = e_ref[...]
  h = gs_ref[...] + gr_ref[...] + b1_ref[...]
  h += jnp.dot(e, w1x_ref[...], preferred_element_type=jnp.float32)
  h = jnp.maximum(h, 0.0)
  en = jnp.dot(h, w2_ref[...], preferred_element_type=jnp.float32) + b2_ref[...]
  enew_ref[...] = en
  eout_ref[...] = en + e


def _node_body(parts_ref, node_ref, w1a_ref, w1b_ref, b1_ref, w2_ref, b2_ref,
               x_ref):
  agg = parts_ref[0] + parts_ref[1]
  agg = agg - jnp.mean(agg, axis=0, keepdims=True)
  node = node_ref[...]
  h = jnp.dot(node, w1a_ref[...], preferred_element_type=jnp.float32)
  h += jnp.dot(agg, w1b_ref[...], preferred_element_type=jnp.float32)
  h = jnp.maximum(h + b1_ref[...], 0.0)
  x = jnp.dot(h, w2_ref[...], preferred_element_type=jnp.float32) + b2_ref[...]
  x = x + node
  x_ref[...] = x - jnp.mean(x, axis=0, keepdims=True)


@functools.cache
def _sc_kernels():
  mesh = plsc.VectorSubcoreMesh(core_axis_name="c", subcore_axis_name="s")

  @functools.partial(
      pl.kernel,
      out_type=(jax.ShapeDtypeStruct((E, D), jnp.float32),
                jax.ShapeDtypeStruct((E, D), jnp.float32)),
      mesh=mesh,
      scratch_types=[
          pltpu.VMEM((B,), jnp.int32),
          pltpu.VMEM((B,), jnp.int32),
          pltpu.VMEM((B, D), jnp.float32),
          pltpu.VMEM((B, D), jnp.float32),
          pltpu.SemaphoreType.DMA,
          pltpu.SemaphoreType.DMA,
      ],
  )
  def sc_gather(ps_hbm, pr_hbm, s_hbm, r_hbm, gs_hbm, gr_hbm,
                si_v, ri_v, rs_v, rr_v, sem_s, sem_r):
    wid = lax.axis_index("s") * NC + lax.axis_index("c")

    def body(k, carry):
      b = wid + NW * k

      @pl.when(b < NB)
      def _():
        off = b * B
        pltpu.sync_copy(s_hbm.at[pl.ds(off, B)], si_v)
        pltpu.sync_copy(r_hbm.at[pl.ds(off, B)], ri_v)
        cp_s = pltpu.async_copy(ps_hbm.at[si_v], rs_v, sem_s)
        cp_r = pltpu.async_copy(pr_hbm.at[ri_v], rr_v, sem_r)
        cp_s.wait()
        cp_r.wait()
        pltpu.sync_copy(rs_v, gs_hbm.at[pl.ds(off, B)])
        pltpu.sync_copy(rr_v, gr_hbm.at[pl.ds(off, B)])

      return carry

    lax.fori_loop(0, ITERS, body, None)

  @functools.partial(
      pl.kernel,
      out_type=jax.ShapeDtypeStruct((NC, N, D), jnp.float32),
      mesh=mesh,
      scratch_types=[
          pltpu.VMEM_SHARED((N, D), jnp.float32),
          pltpu.VMEM((B,), jnp.int32),
          pltpu.VMEM((B, D), jnp.float32),
      ],
  )
  def sc_scatter(enew_hbm, r_hbm, zeros_hbm, out_hbm, acc_sh, ri_v, rows_v):
    c = lax.axis_index("c")
    s = lax.axis_index("s")
    wid = s * NC + c

    @pl.when(s == 0)
    def _():
      pltpu.sync_copy(zeros_hbm, acc_sh)

    plsc.subcore_barrier()

    def body(k, carry):
      b = wid + NW * k

      @pl.when(b < NB)
      def _():
        off = b * B
        pltpu.sync_copy(r_hbm.at[pl.ds(off, B)], ri_v)
        pltpu.sync_copy(enew_hbm.at[pl.ds(off, B)], rows_v)
        pltpu.sync_copy(rows_v, acc_sh.at[ri_v], add=True)

      return carry

    lax.fori_loop(0, ITERS, body, None)
    plsc.subcore_barrier()

    @pl.when(s == 0)
    def _():
      pltpu.sync_copy(acc_sh, out_hbm.at[c])

  return sc_gather, sc_scatter


EB = 3200


def kernel(node_attr, edge_index, edge_attr, W1e, b1e, W2e, b2e,
           W1n, b1n, W2n, b2n):
  senders = edge_index[0]
  receivers = edge_index[1]
  w1s, w1r, w1x = W1e[:D], W1e[D:2 * D], W1e[2 * D:]
  b1e2 = b1e.reshape(1, D)
  b2e2 = b2e.reshape(1, D)
  w1na, w1nb = W1n[:D], W1n[D:]
  b1n2 = b1n.reshape(1, D)
  b2n2 = b2n.reshape(1, D)

  ps, pr = pl.pallas_call(
      _pre_body,
      out_shape=(jax.ShapeDtypeStruct((N, D), jnp.float32),
                 jax.ShapeDtypeStruct((N, D), jnp.float32)),
  )(node_attr, w1s, w1r)

  sc_gather, sc_scatter = _sc_kernels()
  gs, gr = sc_gather(ps, pr, senders, receivers)

  grid = E // EB
  blk = pl.BlockSpec((EB, D), lambda i: (i, 0))
  wspec = pl.BlockSpec((D, D), lambda i: (0, 0))
  bspec = pl.BlockSpec((1, D), lambda i: (0, 0))
  enew, edge_out = pl.pallas_call(
      _edge_body,
      grid=(grid,),
      in_specs=[blk, blk, blk, wspec, bspec, wspec, bspec],
      out_specs=(blk, blk),
      out_shape=(jax.ShapeDtypeStruct((E, D), jnp.float32),
                 jax.ShapeDtypeStruct((E, D), jnp.float32)),
  )(gs, gr, edge_attr, w1x, b1e2, W2e, b2e2)

  zeros = jnp.zeros((N, D), jnp.float32)
  parts = sc_scatter(enew, receivers, zeros)

  x = pl.pallas_call(
      _node_body,
      out_shape=jax.ShapeDtypeStruct((N, D), jnp.float32),
  )(parts, node_attr, w1na, w1nb, b1n2, W2n, b2n2)

  return (x, edge_index, edge_out)
